# baseline (device time: 85721 ns/iter reference)
import jax
import jax.numpy as jnp
from jax import lax
from jax.experimental import pallas as pl
from jax.experimental.pallas import tpu as pltpu

N_DEV = 4


def kernel(x, w_mat):
    m, k_per = x.shape
    _, n = w_mat.shape
    m_chunk = m // N_DEV

    def body(x_ref, w_ref, out_ref, comm_ref, rs_send, rs_recv, ag_send, ag_recv):
        my = lax.axis_index("i")
        left = lax.rem(my + N_DEV - 1, N_DEV)
        right = lax.rem(my + 1, N_DEV)

        barrier = pltpu.get_barrier_semaphore()
        for nbr in (left, right):
            pl.semaphore_signal(
                barrier, inc=1,
                device_id=(nbr,), device_id_type=pl.DeviceIdType.MESH,
            )
        pl.semaphore_wait(barrier, 2)

        out_ref[...] = jnp.dot(
            x_ref[...], w_ref[...], preferred_element_type=jnp.float32
        )

        for s in range(N_DEV - 1):
            send_c = lax.rem(my - s + 2 * N_DEV, N_DEV)
            recv_c = lax.rem(my - s - 1 + 2 * N_DEV, N_DEV)
            rdma = pltpu.make_async_remote_copy(
                src_ref=out_ref.at[pl.ds(send_c * m_chunk, m_chunk), :],
                dst_ref=comm_ref.at[s],
                send_sem=rs_send.at[s],
                recv_sem=rs_recv.at[s],
                device_id=(right,),
                device_id_type=pl.DeviceIdType.MESH,
            )
            rdma.start()
            rdma.wait()
            rows = pl.ds(recv_c * m_chunk, m_chunk)
            out_ref[rows, :] = out_ref[rows, :] + comm_ref[s]

        for s in range(N_DEV - 1):
            send_c = lax.rem(my + 1 - s + 2 * N_DEV, N_DEV)
            rows = pl.ds(send_c * m_chunk, m_chunk)
            rdma = pltpu.make_async_remote_copy(
                src_ref=out_ref.at[rows, :],
                dst_ref=out_ref.at[rows, :],
                send_sem=ag_send.at[s],
                recv_sem=ag_recv.at[s],
                device_id=(right,),
                device_id_type=pl.DeviceIdType.MESH,
            )
            rdma.start()
            rdma.wait()

    return pl.pallas_call(
        body,
        out_shape=jax.ShapeDtypeStruct((m, n), jnp.float32),
        in_specs=[
            pl.BlockSpec(memory_space=pltpu.VMEM),
            pl.BlockSpec(memory_space=pltpu.VMEM),
        ],
        out_specs=pl.BlockSpec(memory_space=pltpu.VMEM),
        scratch_shapes=[
            pltpu.VMEM((N_DEV - 1, m_chunk, n), jnp.float32),
            pltpu.SemaphoreType.DMA((N_DEV - 1,)),
            pltpu.SemaphoreType.DMA((N_DEV - 1,)),
            pltpu.SemaphoreType.DMA((N_DEV - 1,)),
            pltpu.SemaphoreType.DMA((N_DEV - 1,)),
        ],
        compiler_params=pltpu.CompilerParams(collective_id=0),
    )(x, w_mat)


# device time: 52304 ns/iter; 1.6389x vs baseline; 1.6389x over previous
import jax
import jax.numpy as jnp
from jax import lax
from jax.experimental import pallas as pl
from jax.experimental.pallas import tpu as pltpu

N_DEV = 4
R, L = 0, 1


def kernel(x, w_mat):
    m, k_per = x.shape
    _, n = w_mat.shape
    m_chunk = m // N_DEV
    half = n // 2

    cols_r = pl.ds(0, half)
    cols_l = pl.ds(half, half)

    def body(x_ref, w_ref, out_ref, comm_ref, rs_send, rs_recv, ag_send, ag_recv):
        my = lax.axis_index("i")
        left = lax.rem(my + N_DEV - 1, N_DEV)
        right = lax.rem(my + 1, N_DEV)

        barrier = pltpu.get_barrier_semaphore()
        for nbr in (left, right):
            pl.semaphore_signal(
                barrier, inc=1,
                device_id=(nbr,), device_id_type=pl.DeviceIdType.MESH,
            )
        pl.semaphore_wait(barrier, 2)

        out_ref[...] = jnp.dot(
            x_ref[...], w_ref[...], preferred_element_type=jnp.float32
        )

        def rows(c):
            return pl.ds(c * m_chunk, m_chunk)

        for s in range(N_DEV - 1):
            send_r = lax.rem(my - s + 2 * N_DEV, N_DEV)
            recv_r = lax.rem(my - s - 1 + 2 * N_DEV, N_DEV)
            send_l = lax.rem(my + s, N_DEV)
            recv_l = lax.rem(my + s + 1, N_DEV)
            rdma_r = pltpu.make_async_remote_copy(
                src_ref=out_ref.at[rows(send_r), cols_r],
                dst_ref=comm_ref.at[s, :, cols_r],
                send_sem=rs_send.at[s, R],
                recv_sem=rs_recv.at[s, R],
                device_id=(right,),
                device_id_type=pl.DeviceIdType.MESH,
            )
            rdma_l = pltpu.make_async_remote_copy(
                src_ref=out_ref.at[rows(send_l), cols_l],
                dst_ref=comm_ref.at[s, :, cols_l],
                send_sem=rs_send.at[s, L],
                recv_sem=rs_recv.at[s, L],
                device_id=(left,),
                device_id_type=pl.DeviceIdType.MESH,
            )
            rdma_r.start()
            rdma_l.start()
            rdma_r.wait()
            rdma_l.wait()
            out_ref[rows(recv_r), cols_r] = (
                out_ref[rows(recv_r), cols_r] + comm_ref[s, :, cols_r]
            )
            out_ref[rows(recv_l), cols_l] = (
                out_ref[rows(recv_l), cols_l] + comm_ref[s, :, cols_l]
            )

        for s in range(N_DEV - 1):
            send_r = lax.rem(my + 1 - s + 2 * N_DEV, N_DEV)
            send_l = lax.rem(my - 1 + s + N_DEV, N_DEV)
            rdma_r = pltpu.make_async_remote_copy(
                src_ref=out_ref.at[rows(send_r), cols_r],
                dst_ref=out_ref.at[rows(send_r), cols_r],
                send_sem=ag_send.at[s, R],
                recv_sem=ag_recv.at[s, R],
                device_id=(right,),
                device_id_type=pl.DeviceIdType.MESH,
            )
            rdma_l = pltpu.make_async_remote_copy(
                src_ref=out_ref.at[rows(send_l), cols_l],
                dst_ref=out_ref.at[rows(send_l), cols_l],
                send_sem=ag_send.at[s, L],
                recv_sem=ag_recv.at[s, L],
                device_id=(left,),
                device_id_type=pl.DeviceIdType.MESH,
            )
            rdma_r.start()
            rdma_l.start()
            rdma_r.wait()
            rdma_l.wait()

    return pl.pallas_call(
        body,
        out_shape=jax.ShapeDtypeStruct((m, n), jnp.float32),
        in_specs=[
            pl.BlockSpec(memory_space=pltpu.VMEM),
            pl.BlockSpec(memory_space=pltpu.VMEM),
        ],
        out_specs=pl.BlockSpec(memory_space=pltpu.VMEM),
        scratch_shapes=[
            pltpu.VMEM((N_DEV - 1, m_chunk, n), jnp.float32),
            pltpu.SemaphoreType.DMA((N_DEV - 1, 2)),
            pltpu.SemaphoreType.DMA((N_DEV - 1, 2)),
            pltpu.SemaphoreType.DMA((N_DEV - 1, 2)),
            pltpu.SemaphoreType.DMA((N_DEV - 1, 2)),
        ],
        compiler_params=pltpu.CompilerParams(collective_id=0),
    )(x, w_mat)


# device time: 44206 ns/iter; 1.9391x vs baseline; 1.1832x over previous
import jax
import jax.numpy as jnp
from jax import lax
from jax.experimental import pallas as pl
from jax.experimental.pallas import tpu as pltpu

N_DEV = 4
SUB = 4
R, L = 0, 1


def kernel(x, w_mat):
    m, k_per = x.shape
    _, n = w_mat.shape
    m_chunk = m // N_DEV
    half = n // 2
    subw = half // SUB

    def col(d, j):
        return pl.ds(d * half + j * subw, subw)

    def body(x_ref, w_ref, out_ref, comm_ref, rs_send, rs_recv, ag_send, ag_recv):
        my = lax.axis_index("i")
        left = lax.rem(my + N_DEV - 1, N_DEV)
        right = lax.rem(my + 1, N_DEV)

        barrier = pltpu.get_barrier_semaphore()
        for nbr in (left, right):
            pl.semaphore_signal(
                barrier, inc=1,
                device_id=(nbr,), device_id_type=pl.DeviceIdType.MESH,
            )
        pl.semaphore_wait(barrier, 2)

        def rows(c):
            return pl.ds(lax.rem(c + 4 * N_DEV, N_DEV) * m_chunk, m_chunk)

        def rs_send_chunk(d, s):
            return my - s if d == R else my + s

        def rs_recv_chunk(d, s):
            return my - s - 1 if d == R else my + s + 1

        def ag_chunk(d, s):
            return my + 1 - s if d == R else my - 1 + s

        tgt = {R: right, L: left}

        def make(src, dst, sems_s, sems_r, s, d, j):
            return pltpu.make_async_remote_copy(
                src_ref=src,
                dst_ref=dst,
                send_sem=sems_s.at[s, d, j],
                recv_sem=sems_r.at[s, d, j],
                device_id=(tgt[d],),
                device_id_type=pl.DeviceIdType.MESH,
            )

        rs = {
            (s, d, j): make(
                out_ref.at[rows(rs_send_chunk(d, s)), col(d, j)],
                comm_ref.at[s, :, col(d, j)],
                rs_send, rs_recv, s, d, j,
            )
            for s in range(N_DEV - 1) for d in (R, L) for j in range(SUB)
        }
        ag = {
            (s, d, j): make(
                out_ref.at[rows(ag_chunk(d, s)), col(d, j)],
                out_ref.at[rows(ag_chunk(d, s)), col(d, j)],
                ag_send, ag_recv, s, d, j,
            )
            for s in range(N_DEV - 1) for d in (R, L) for j in range(SUB)
        }

        def acc(s, j):
            for d in (R, L):
                rws = rows(rs_recv_chunk(d, s))
                out_ref[rws, col(d, j)] = (
                    out_ref[rws, col(d, j)] + comm_ref[s, :, col(d, j)]
                )

        def gemm_chunk(c):
            rws = rows(c)
            out_ref[rws, :] = jnp.dot(
                x_ref[rws, :], w_ref[...], preferred_element_type=jnp.float32
            )

        gemm_chunk(my)
        for j in range(SUB):
            rs[(0, R, j)].start()
            rs[(0, L, j)].start()
        for t in (-1, 1, 2):
            gemm_chunk(my + t)

        for s in range(1, N_DEV - 1):
            for j in range(SUB):
                rs[(s - 1, R, j)].wait_recv()
                rs[(s - 1, L, j)].wait_recv()
                acc(s - 1, j)
                rs[(s, R, j)].start()
                rs[(s, L, j)].start()
        for j in range(SUB):
            rs[(N_DEV - 2, R, j)].wait_recv()
            rs[(N_DEV - 2, L, j)].wait_recv()
            acc(N_DEV - 2, j)
            ag[(0, R, j)].start()
            ag[(0, L, j)].start()
        for s in range(1, N_DEV - 1):
            for j in range(SUB):
                ag[(s - 1, R, j)].wait_recv()
                ag[(s - 1, L, j)].wait_recv()
                ag[(s, R, j)].start()
                ag[(s, L, j)].start()
        for j in range(SUB):
            ag[(N_DEV - 2, R, j)].wait_recv()
            ag[(N_DEV - 2, L, j)].wait_recv()

        for desc in list(rs.values()) + list(ag.values()):
            desc.wait_send()

    return pl.pallas_call(
        body,
        out_shape=jax.ShapeDtypeStruct((m, n), jnp.float32),
        in_specs=[
            pl.BlockSpec(memory_space=pltpu.VMEM),
            pl.BlockSpec(memory_space=pltpu.VMEM),
        ],
        out_specs=pl.BlockSpec(memory_space=pltpu.VMEM),
        scratch_shapes=[
            pltpu.VMEM((N_DEV - 1, m_chunk, n), jnp.float32),
            pltpu.SemaphoreType.DMA((N_DEV - 1, 2, SUB)),
            pltpu.SemaphoreType.DMA((N_DEV - 1, 2, SUB)),
            pltpu.SemaphoreType.DMA((N_DEV - 1, 2, SUB)),
            pltpu.SemaphoreType.DMA((N_DEV - 1, 2, SUB)),
        ],
        compiler_params=pltpu.CompilerParams(collective_id=0),
    )(x, w_mat)
